# baseline (speedup 1.0000x reference)
import functools

import jax
import jax.numpy as jnp
from jax import lax
from jax.experimental import pallas as pl
from jax.experimental.pallas import tpu as pltpu
from jax.experimental.pallas import tpu_sc as plsc

N = 100000
E = 1600000
NC = 2
NS = 16
NW = NC * NS
LANE = 16

N_PAD = 102400
ROW = 128
ROWS_PER_W = 392
E_PAD = NW * ROWS_PER_W * ROW
ROWS_TOTAL = E_PAD // ROW
NSL = N_PAD // NS

_mesh = plsc.VectorSubcoreMesh(core_axis_name="c", subcore_axis_name="s")
_sc_params = pltpu.CompilerParams(use_tc_tiling_on_sc=False)
_f32 = jnp.float32


def _worker(c, s):
    return c * NS + s


_DEG_K = 28


@functools.partial(
    pl.kernel,
    out_type=jax.ShapeDtypeStruct((NC, N_PAD), _f32),
    mesh=_mesh,
    compiler_params=_sc_params,
    scratch_types=[
        pltpu.VMEM((2, _DEG_K, ROW), jnp.int32),
        pltpu.VMEM((ROW,), _f32),
        pltpu.VMEM((ROW,), _f32),
        pltpu.VMEM_SHARED((N_PAD,), _f32),
        pltpu.SemaphoreType.DMA,
        pltpu.SemaphoreType.DMA,
    ],
)
def _sc_degree(dst_hbm, out_hbm, didx_v, ones_v, zeros_v, acc, sem_i, sem_s):
    K = _DEG_K
    n_chunks = ROWS_PER_W // K
    c = lax.axis_index("c")
    s = lax.axis_index("s")
    w = _worker(c, s)
    for i in range(ROW // LANE):
        ones_v[pl.ds(i * LANE, LANE)] = jnp.ones((LANE,), _f32)
        zeros_v[pl.ds(i * LANE, LANE)] = jnp.zeros((LANE,), _f32)

    def zero_body(i, _):
        pltpu.sync_copy(zeros_v, acc.at[pl.ds(s * NSL + i * ROW, ROW)])
        return 0

    lax.fori_loop(0, NSL // ROW, zero_body, 0)
    plsc.subcore_barrier()

    def rows(ch):
        return pl.ds(w * ROWS_PER_W + ch * K, K)

    pltpu.make_async_copy(dst_hbm.at[rows(0), :], didx_v.at[0], sem_i).start()

    def body(ch, _):
        b = lax.rem(ch, 2)
        pltpu.make_async_copy(dst_hbm.at[rows(ch), :], didx_v.at[b], sem_i).wait()

        for j in range(K):
            pltpu.make_async_copy(ones_v, acc.at[didx_v.at[b, j]], sem_s).start(add=True)

        @pl.when(ch > 0)
        def _():
            for j in range(K):
                pltpu.make_async_copy(ones_v, acc.at[didx_v.at[1 - b, j]], sem_s).wait()

        @pl.when(ch + 1 < n_chunks)
        def _():
            pltpu.make_async_copy(
                dst_hbm.at[rows(ch + 1), :], didx_v.at[1 - b], sem_i).start()

        return 0

    lax.fori_loop(0, n_chunks, body, 0)
    last = (n_chunks - 1) % 2
    for j in range(K):
        pltpu.make_async_copy(ones_v, acc.at[didx_v.at[last, j]], sem_s).wait()
    plsc.subcore_barrier()
    pltpu.sync_copy(acc.at[pl.ds(s * NSL, NSL)], out_hbm.at[c, pl.ds(s * NSL, NSL)])


def _make_agg(F, K):
    assert ROWS_PER_W % K == 0
    n_chunks = ROWS_PER_W // K
    vshape = (N_PAD,) if F == 1 else (N_PAD, F)
    scratch = [
        pltpu.VMEM((2, K, ROW), jnp.int32),
        pltpu.VMEM((2, K, ROW), jnp.int32),
        pltpu.VMEM((2, K * ROW) if F == 1 else (2, K * ROW, F), _f32),
    ]
    if F == 1:
        scratch.append(pltpu.VMEM_SHARED((N_PAD,), _f32))
    scratch += [
        pltpu.VMEM_SHARED(vshape, _f32),
        pltpu.SemaphoreType.DMA,
        pltpu.SemaphoreType.DMA,
        pltpu.SemaphoreType.DMA,
    ]

    @functools.partial(
        pl.kernel,
        out_type=jax.ShapeDtypeStruct((NC,) + vshape, _f32),
        mesh=_mesh,
        compiler_params=_sc_params,
        scratch_types=scratch,
    )
    def agg(src_hbm, dst_hbm, u_hbm, out_hbm, *rest):
        if F == 1:
            sidx_v, didx_v, vals_v, tbl, acc, sem_i, sem_g, sem_s = rest
        else:
            sidx_v, didx_v, vals_v, acc, sem_i, sem_g, sem_s = rest
        c = lax.axis_index("c")
        s = lax.axis_index("s")
        w = _worker(c, s)
        sl = pl.ds(s * NSL, NSL)

        def nodes(ref, a=sl):
            return ref.at[a] if F == 1 else ref.at[a, :]

        def vsl(bank, j):
            a = pl.ds(j * ROW, ROW)
            return vals_v.at[bank, a] if F == 1 else vals_v.at[bank, a, :]

        gsrc = tbl if F == 1 else u_hbm

        if F == 1:
            pltpu.sync_copy(u_hbm.at[sl], tbl.at[sl])
        pltpu.sync_copy(nodes(u_hbm), nodes(acc))
        plsc.subcore_barrier()

        def rows(ch):
            return pl.ds(w * ROWS_PER_W + ch * K, K)

        pltpu.make_async_copy(src_hbm.at[rows(0), :], sidx_v.at[0], sem_i).start()
        pltpu.make_async_copy(dst_hbm.at[rows(0), :], didx_v.at[0], sem_i).start()

        def body(ch, _):
            b = lax.rem(ch, 2)
            pltpu.make_async_copy(src_hbm.at[rows(ch), :], sidx_v.at[b], sem_i).wait()
            pltpu.make_async_copy(dst_hbm.at[rows(ch), :], didx_v.at[b], sem_i).wait()

            for j in range(K):
                pltpu.make_async_copy(
                    gsrc.at[sidx_v.at[b, j]], vsl(b, j), sem_g).start()

            @pl.when(ch > 0)
            def _():
                for j in range(K):
                    pltpu.make_async_copy(
                        vsl(1 - b, j), acc.at[didx_v.at[1 - b, j]], sem_s).wait()

            @pl.when(ch + 1 < n_chunks)
            def _():
                pltpu.make_async_copy(
                    src_hbm.at[rows(ch + 1), :], sidx_v.at[1 - b], sem_i).start()
                pltpu.make_async_copy(
                    dst_hbm.at[rows(ch + 1), :], didx_v.at[1 - b], sem_i).start()

            for j in range(K):
                pltpu.make_async_copy(
                    gsrc.at[sidx_v.at[b, j]], vsl(b, j), sem_g).wait()
            for j in range(K):
                pltpu.make_async_copy(
                    vsl(b, j), acc.at[didx_v.at[b, j]], sem_s).start(add=True)

            return 0

        lax.fori_loop(0, n_chunks, body, 0)
        last = (n_chunks - 1) % 2
        for j in range(K):
            pltpu.make_async_copy(
                vsl(last, j), acc.at[didx_v.at[last, j]], sem_s).wait()
        plsc.subcore_barrier()
        if F == 1:
            pltpu.sync_copy(acc.at[sl], out_hbm.at[c, sl])
        else:
            pltpu.sync_copy(acc.at[sl, :], out_hbm.at[c, sl, :])

    return agg


_sc_agg1 = _make_agg(1, 28)
_sc_agg16 = _make_agg(16, 4)


_BLK = 2048
_GRID = N_PAD // _BLK


def _col_spec():
    return pl.BlockSpec((_BLK, 1), lambda g: (g, 0))


def _tc_b_body(d0, d1, x, dinv, u1):
    deg = d0[...] + d1[...] + 1.0
    di = lax.rsqrt(deg)
    dinv[...] = di
    u1[...] = x[...] * di


def _tc_d_body(p0, p1, u1, dinv, W1, b1, W2, u2):
    y1 = (p0[...] + p1[...] - u1[...]) * dinv[...]
    h1 = jax.nn.relu(y1 * W1[...] + b1[...])
    z = jnp.dot(h1, W2[...], preferred_element_type=jnp.float32)
    u2[...] = z * dinv[...]


def _tc_f_body(q0, q1, u2, dinv, b2, W3, u3):
    h2 = jax.nn.relu((q0[...] + q1[...] - u2[...]) * dinv[...] + b2[...])
    wv = jnp.dot(h2, W3[...], preferred_element_type=jnp.float32)
    u3[...] = wv * dinv[...]


def _tc_h_body(r0, r1, u3, dinv, b3, out):
    out[...] = (r0[...] + r1[...] - u3[...]) * dinv[...] + b3[0, 0]


def kernel(x, edge_index, W1, b1, W2, b2, W3, b3):
    f32 = jnp.float32
    ei = edge_index.astype(jnp.int32)
    n_extra = E_PAD - E
    pad_idx = N + (jnp.arange(n_extra, dtype=jnp.int32) % (N_PAD - N - 8))
    src2d = jnp.concatenate([ei[0], pad_idx]).reshape(ROWS_TOTAL, ROW)
    dst2d = jnp.concatenate([ei[1], pad_idx]).reshape(ROWS_TOTAL, ROW)
    x_pad = jnp.pad(x, ((0, N_PAD - N), (0, 0)))

    degp = _sc_degree(dst2d)
    d0 = degp[0].reshape(N_PAD, 1)
    d1 = degp[1].reshape(N_PAD, 1)

    dinv, u1 = pl.pallas_call(
        _tc_b_body,
        grid=(_GRID,),
        in_specs=[_col_spec(), _col_spec(), _col_spec()],
        out_specs=[_col_spec(), _col_spec()],
        out_shape=[jax.ShapeDtypeStruct((N_PAD, 1), f32)] * 2,
    )(d0, d1, x_pad)

    s1 = _sc_agg1(src2d, dst2d, u1.reshape(N_PAD))
    p0 = s1[0].reshape(N_PAD, 1)
    p1 = s1[1].reshape(N_PAD, 1)

    u2 = pl.pallas_call(
        _tc_d_body,
        grid=(_GRID,),
        in_specs=[
            _col_spec(), _col_spec(), _col_spec(), _col_spec(),
            pl.BlockSpec((1, 32), lambda g: (0, 0)),
            pl.BlockSpec((1, 32), lambda g: (0, 0)),
            pl.BlockSpec((32, 16), lambda g: (0, 0)),
        ],
        out_specs=pl.BlockSpec((_BLK, 16), lambda g: (g, 0)),
        out_shape=jax.ShapeDtypeStruct((N_PAD, 16), f32),
    )(p0, p1, u1, dinv, W1, b1.reshape(1, 32), W2)

    s2 = _sc_agg16(src2d, dst2d, u2)

    u3 = pl.pallas_call(
        _tc_f_body,
        grid=(_GRID,),
        in_specs=[
            pl.BlockSpec((_BLK, 16), lambda g: (g, 0)),
            pl.BlockSpec((_BLK, 16), lambda g: (g, 0)),
            pl.BlockSpec((_BLK, 16), lambda g: (g, 0)),
            _col_spec(),
            pl.BlockSpec((1, 16), lambda g: (0, 0)),
            pl.BlockSpec((16, 1), lambda g: (0, 0)),
        ],
        out_specs=_col_spec(),
        out_shape=jax.ShapeDtypeStruct((N_PAD, 1), f32),
    )(s2[0], s2[1], u2, dinv, b2.reshape(1, 16), W3)

    s3 = _sc_agg1(src2d, dst2d, u3.reshape(N_PAD))
    r0 = s3[0].reshape(N_PAD, 1)
    r1 = s3[1].reshape(N_PAD, 1)

    out = pl.pallas_call(
        _tc_h_body,
        grid=(_GRID,),
        in_specs=[
            _col_spec(), _col_spec(), _col_spec(), _col_spec(),
            pl.BlockSpec((1, 1), lambda g: (0, 0)),
        ],
        out_specs=_col_spec(),
        out_shape=jax.ShapeDtypeStruct((N_PAD, 1), f32),
    )(r0, r1, u3, dinv, b3.reshape(1, 1))

    return out[:N]
